# baseline (speedup 1.0000x reference)
import functools

import jax
import jax.numpy as jnp
from jax import lax
from jax.experimental import pallas as pl
from jax.experimental.pallas import tpu as pltpu
from jax.experimental.pallas import tpu_sc as plsc

NQ = 1024
NBANK = 100000
D = 16
NCLS = 100
NBINS = 1024
LANES = 16
NC = 2
NS = 16
NTEC = NC * NS
ROWS_PER_TEC = NQ // NTEC

SIMS_N = 100352
CHUNK = 128
NCHUNK = SIMS_N // CHUNK
QT = 1024
BT = 2048
CPB = BT // CHUNK
NBLKJ = SIMS_N // BT
PAD_LOCAL = NBANK - (NBLKJ - 1) * BT
NEG = -3e38
GCH = 16
WGR = 16
IDXBUF = 896


def _logsumexp_rows(x):
    m = jnp.max(x, axis=1, keepdims=True)
    return jnp.log(jnp.sum(jnp.exp(x - m), axis=1, keepdims=True)) + m


def _sims_body(qt, feat_ref, logit_ref, blog_ref, bfeas_ref, sims_ref,
               cmax_ref, rmin_ref, rmax_ref, energy_ref):
    lse = _logsumexp_rows(blog_ref[...])
    bf = bfeas_ref[...]
    bnorm = jnp.sqrt(jnp.sum(bf * bf, axis=1, keepdims=True))
    g = (bf / bnorm + 1e-10) * lse
    f = feat_ref[...]
    norm = jnp.sqrt(jnp.sum(f * f, axis=1, keepdims=True))
    fn = f / norm + 1e-10
    s = lax.dot_general(fn, g, (((1,), (1,)), ((), ())),
                        preferred_element_type=jnp.float32)
    j = pl.program_id(1)

    def emit(s_out, s_for_min):
        s3 = s_out.reshape(qt, CPB, CHUNK)
        sims_ref[...] = s3
        cmax_ref[...] = jnp.max(s3, axis=2).reshape(1, qt, CPB)
        pmin = jnp.min(s_for_min, axis=1, keepdims=True)
        pmax = jnp.max(s_out, axis=1, keepdims=True)
        return pmin, pmax

    @pl.when(j == 0)
    def _():
        pmin, pmax = emit(s, s)
        rmin_ref[...] = pmin
        rmax_ref[...] = pmax
        energy_ref[...] = _logsumexp_rows(logit_ref[...])

    @pl.when(jnp.logical_and(j != 0, j != NBLKJ - 1))
    def _():
        pmin, pmax = emit(s, s)
        rmin_ref[...] = jnp.minimum(rmin_ref[...], pmin)
        rmax_ref[...] = jnp.maximum(rmax_ref[...], pmax)

    @pl.when(j == NBLKJ - 1)
    def _():
        lcol = lax.broadcasted_iota(jnp.int32, (qt, BT), 1)
        pad = lcol >= PAD_LOCAL
        pmin, pmax = emit(jnp.where(pad, NEG, s), jnp.where(pad, 3e38, s))
        rmin_ref[...] = jnp.minimum(rmin_ref[...], pmin)
        rmax_ref[...] = jnp.maximum(rmax_ref[...], pmax)


def _sims_stage(feature, logit, bank_feas, bank_logits):
    nq = feature.shape[0]
    qt = min(QT, nq)
    return pl.pallas_call(
        functools.partial(_sims_body, qt),
        grid=(nq // qt, NBLKJ),
        in_specs=[
            pl.BlockSpec((qt, D), lambda q, j: (q, 0)),
            pl.BlockSpec((qt, NCLS), lambda q, j: (q, 0)),
            pl.BlockSpec((BT, NCLS), lambda q, j: (j, 0)),
            pl.BlockSpec((BT, D), lambda q, j: (j, 0)),
        ],
        out_specs=[
            pl.BlockSpec((qt, CPB, CHUNK), lambda q, j: (q, j, 0)),
            pl.BlockSpec((1, qt, CPB), lambda q, j: (j, q, 0)),
            pl.BlockSpec((qt, 1), lambda q, j: (q, 0)),
            pl.BlockSpec((qt, 1), lambda q, j: (q, 0)),
            pl.BlockSpec((qt, 1), lambda q, j: (q, 0)),
        ],
        out_shape=[
            jax.ShapeDtypeStruct((nq, NCHUNK, CHUNK), jnp.float32),
            jax.ShapeDtypeStruct((NBLKJ, nq, CPB), jnp.float32),
            jax.ShapeDtypeStruct((nq, 1), jnp.float32),
            jax.ShapeDtypeStruct((nq, 1), jnp.float32),
            jax.ShapeDtypeStruct((nq, 1), jnp.float32),
        ],
    )(feature, logit, bank_logits, bank_feas)


def _suffix_select(hcnt, hsum, target):
    lane_f = lax.iota(jnp.int32, LANES).astype(jnp.float32)
    with_sum = hsum is not None

    def cond(carry):
        return jnp.logical_and(jnp.logical_not(carry[3]), carry[0] >= 0)

    def body(carry):
        j, r_c, r_s, done = carry
        c = hcnt[pl.ds(j * LANES, LANES)]
        tot_c = jnp.sum(c)
        cross = r_c + tot_c >= target
        if with_sum:
            tot_s = jnp.sum(hsum[pl.ds(j * LANES, LANES)])
            r_s = jnp.where(cross, r_s, r_s + tot_s)
        return (jnp.where(cross, j, j - 1),
                jnp.where(cross, r_c, r_c + tot_c), r_s, cross)

    init = (jnp.int32(NBINS // LANES - 1), jnp.float32(0.0), jnp.float32(0.0),
            False)
    j, r_c, r_s, _ = lax.while_loop(cond, body, init)

    j = jnp.maximum(j, 0)
    c = hcnt[pl.ds(j * LANES, LANES)]
    rc = lax.rev(jnp.cumsum(lax.rev(c, (0,))), (0,)) + r_c
    m = rc >= target
    lane = jnp.sum(jnp.where(m, 1.0, 0.0)) - 1.0
    sel = lane_f == lane
    c_l = jnp.sum(jnp.where(sel, c, 0.0))
    rc_l = jnp.sum(jnp.where(sel, rc, 0.0))
    b_sel = (j * LANES).astype(jnp.float32) + lane
    cc = rc_l - c_l
    ss = jnp.float32(0.0)
    if with_sum:
        s = hsum[pl.ds(j * LANES, LANES)]
        rs = lax.rev(jnp.cumsum(lax.rev(s, (0,))), (0,)) + r_s
        s_l = jnp.sum(jnp.where(sel, s, 0.0))
        rs_l = jnp.sum(jnp.where(sel, rs, 0.0))
        ss = rs_l - s_l
    return b_sel, cc, ss


def _scalar_at(ref, i, lane_i):
    vbase = (i // LANES) * LANES
    vec = ref[pl.ds(vbase, LANES)]
    sel = lane_i == (i - vbase)
    return jnp.sum(jnp.where(sel, vec, 0.0))


def _sc_topk_body(k_sel, rpt, sims2_hbm, cmax_hbm, lo_hbm, scale_hbm, w1_hbm,
                  esc_hbm, out_hbm,
                  cm_v, idx_v, cand_a, hcnt, hsum, acc_v,
                  lo_v, scale_v, w1_v, esc_v, res_v, sem_a):
    wid = lax.axis_index("s") * NC + lax.axis_index("c")
    base = wid * rpt
    pltpu.sync_copy(lo_hbm.at[pl.ds(base, rpt)], lo_v)
    pltpu.sync_copy(scale_hbm.at[pl.ds(base, rpt)], scale_v)
    pltpu.sync_copy(w1_hbm.at[pl.ds(base, rpt)], w1_v)
    pltpu.sync_copy(esc_hbm.at[pl.ds(base, rpt)], esc_v)
    ones = jnp.full((LANES,), 1.0, jnp.float32)
    zeros = jnp.zeros((LANES,), jnp.float32)
    izeros = jnp.zeros((LANES,), jnp.int32)
    lane_i = lax.iota(jnp.int32, LANES)
    kf = jnp.float32(k_sel)

    @plsc.parallel_loop(0, IDXBUF // LANES, unroll=5)
    def _init_idx(i):
        idx_v[pl.ds(i * LANES, LANES)] = izeros

    def zero_cnt():
        @plsc.parallel_loop(0, NBINS // LANES, unroll=8)
        def _z(i):
            hcnt[pl.ds(i * LANES, LANES)] = zeros

    def row_body(r, carry):
        q = base + r
        pltpu.sync_copy(cmax_hbm.at[:, q], cm_v)
        lo = _scalar_at(lo_v, r, lane_i)
        scale1 = _scalar_at(scale_v, r, lane_i)

        zero_cnt()

        @plsc.parallel_loop(0, NCHUNK // LANES, unroll=7)
        def _pa(i):
            v = cm_v[i, pl.ds(0, LANES)]
            x = jnp.clip((v - lo) * scale1, 0.0, float(NBINS - 1))
            plsc.addupdate_scatter(hcnt, [x.astype(jnp.int32)], ones)

        bA, _, _ = _suffix_select(hcnt, None, kf)
        bAi = bA.astype(jnp.int32)

        def comp(i, off):
            v = cm_v[i, pl.ds(0, LANES)]
            x = jnp.clip((v - lo) * scale1, 0.0, float(NBINS - 1))
            m = x.astype(jnp.int32) >= bAi
            ids = (q * NCHUNK + i * LANES) + lane_i
            plsc.store_compressed(idx_v.at[pl.ds(off, LANES)], ids, mask=m)
            cnt = plsc.all_reduce_population_count(m)
            return off + cnt[0]

        n_surv = lax.fori_loop(0, NCHUNK // LANES, comp, jnp.int32(0))
        nvals = n_surv * CHUNK
        ngr = (n_surv + jnp.int32(GCH - 1)) >> 4

        def wave_fire(wbase, gcount):
            def fire(g, c):
                idxg = idx_v[pl.ds((wbase + g) * LANES, LANES)]
                pltpu.make_async_copy(
                    sims2_hbm.at[idxg],
                    cand_a.at[pl.ds(g * LANES, LANES)], sem_a).start()
                return c

            lax.fori_loop(0, gcount, fire, 0)

            def drain(g, c):
                idxg = idx_v[pl.ds((wbase + g) * LANES, LANES)]
                pltpu.make_async_copy(
                    sims2_hbm.at[idxg],
                    cand_a.at[pl.ds(g * LANES, LANES)], sem_a).wait()
                return c

            lax.fori_loop(0, gcount, drain, 0)

        def proc_wave(wbase, gcount, body):
            wv = wbase * (LANES * CHUNK)

            @plsc.parallel_loop(0, gcount * (GCH * CHUNK // LANES), unroll=8)
            def _p(i):
                row = i >> 3
                col = (i & 7) * LANES
                v = cand_a[row, pl.ds(col, LANES)]
                valm = (wv + i * LANES + lane_i) < nvals
                body(v, valm)

        nwav = (ngr + jnp.int32(WGR - 1)) >> 4

        def waves(body):
            def wv(w, c):
                wbase = w * WGR
                gcount = jnp.minimum(ngr - wbase, WGR)
                wave_fire(wbase, gcount)
                proc_wave(wbase, gcount, body)
                return c

            lax.fori_loop(0, nwav, wv, 0)

        zero_cnt()

        def body_b(v, valm):
            x = jnp.clip((v - lo) * scale1, 0.0, float(NBINS - 1))
            plsc.addupdate_scatter(hcnt, [x.astype(jnp.int32)], ones,
                                   mask=valm)

        waves(body_b)
        b1, cc1, _ = _suffix_select(hcnt, None, kf)
        w1 = _scalar_at(w1_v, r, lane_i)
        blo = lo + b1 * w1
        scale2 = scale1 * jnp.float32(NBINS)
        k1 = kf - cc1
        b1i = b1.astype(jnp.int32)

        zero_cnt()

        @plsc.parallel_loop(0, NBINS // LANES, unroll=8)
        def _zs(i):
            hsum[pl.ds(i * LANES, LANES)] = zeros

        acc_v[pl.ds(0, LANES)] = zeros

        def body_c(v, valm):
            x = jnp.clip((v - lo) * scale1, 0.0, float(NBINS - 1))
            idx1 = x.astype(jnp.int32)
            plsc.addupdate_scatter(
                acc_v, [lane_i],
                jnp.where(jnp.logical_and(valm, idx1 > b1i), v, 0.0))
            msk = jnp.logical_and(valm, idx1 == b1i)
            x2 = jnp.clip((v - blo) * scale2, 0.0, float(NBINS - 1))
            idx2 = x2.astype(jnp.int32)
            plsc.addupdate_scatter(hcnt, [idx2], ones, mask=msk)
            plsc.addupdate_scatter(hsum, [idx2], v, mask=msk)

        @pl.when(nwav == 1)
        def _():
            proc_wave(0, ngr, body_c)

        @pl.when(nwav > 1)
        def _():
            waves(body_c)

        s1 = jnp.sum(acc_v[pl.ds(0, LANES)])
        b2, cc2, s2 = _suffix_select(hcnt, hsum, k1)
        t_hat = blo + b2 * (w1 * jnp.float32(1.0 / NBINS))
        tsum = s1 + s2 + (k1 - cc2) * t_hat
        res = tsum * _scalar_at(esc_v, r, lane_i)
        vbase = (r // LANES) * LANES
        sel = lane_i == (r - vbase)
        plsc.store_scatter(res_v, [jnp.full((LANES,), vbase, jnp.int32) + lane_i],
                           jnp.full((LANES,), 1.0, jnp.float32) * res, mask=sel)
        return carry

    lax.fori_loop(0, rpt, row_body, 0)
    pltpu.sync_copy(res_v, out_hbm.at[pl.ds(base, rpt)])


def _sc_topk(sims2, cmax, lo, scale1, w1, esc, k_sel):
    nq = lo.shape[0]
    rpt = nq // NTEC
    mesh = plsc.VectorSubcoreMesh(core_axis_name="c", subcore_axis_name="s")
    fn = pl.kernel(
        functools.partial(_sc_topk_body, k_sel, rpt),
        mesh=mesh,
        compiler_params=pltpu.CompilerParams(needs_layout_passes=False),
        out_type=jax.ShapeDtypeStruct((nq,), jnp.float32),
        scratch_types=[
            pltpu.VMEM((NBLKJ, CPB), jnp.float32),
            pltpu.VMEM((IDXBUF,), jnp.int32),
            pltpu.VMEM((WGR * GCH, CHUNK), jnp.float32),
            pltpu.VMEM((NBINS,), jnp.float32),
            pltpu.VMEM((NBINS,), jnp.float32),
            pltpu.VMEM((LANES,), jnp.float32),
            pltpu.VMEM((rpt,), jnp.float32),
            pltpu.VMEM((rpt,), jnp.float32),
            pltpu.VMEM((rpt,), jnp.float32),
            pltpu.VMEM((rpt,), jnp.float32),
            pltpu.VMEM((rpt,), jnp.float32),
            pltpu.SemaphoreType.DMA,
        ],
    )
    return fn(sims2, cmax, lo, scale1, w1, esc)


def kernel(feature, logit, bank_feas, bank_logits, k):
    k_sel = logit.shape[-1]
    nsplit = 2
    nqh = NQ // nsplit
    outs = []
    for p in range(nsplit):
        fh = feature[p * nqh:(p + 1) * nqh]
        lh = logit[p * nqh:(p + 1) * nqh]
        sims3, cmax, rmin, rmax, energy = _sims_stage(fh, lh, bank_feas,
                                                      bank_logits)
        sims2 = sims3.reshape(nqh * NCHUNK, CHUNK)
        lo = rmin.reshape(nqh)
        span = jnp.maximum(rmax.reshape(nqh) - lo, 1e-30)
        scale1 = jnp.float32(NBINS) / span
        w1 = span * jnp.float32(1.0 / NBINS)
        esc = -energy.reshape(nqh) / k
        outs.append(_sc_topk(sims2, cmax, lo, scale1, w1, esc, k_sel))
    return jnp.concatenate(outs)


# speedup vs baseline: 1.1372x; 1.1372x over previous
import functools

import jax
import jax.numpy as jnp
from jax import lax
from jax.experimental import pallas as pl
from jax.experimental.pallas import tpu as pltpu
from jax.experimental.pallas import tpu_sc as plsc

NQ = 1024
NBANK = 100000
D = 16
NCLS = 100
NBINS = 1024
LANES = 16
NC = 2
NS = 16
NTEC = NC * NS
ROWS_PER_TEC = NQ // NTEC

SIMS_N = 100352
CHUNK = 128
NCHUNK = SIMS_N // CHUNK
QT = 1024
BT = 2048
CPB = BT // CHUNK
NBLKJ = SIMS_N // BT
PAD_LOCAL = NBANK - (NBLKJ - 1) * BT
NEG = -3e38
GCH = 16
WGR = 16
IDXBUF = 896


def _logsumexp_rows(x):
    m = jnp.max(x, axis=1, keepdims=True)
    return jnp.log(jnp.sum(jnp.exp(x - m), axis=1, keepdims=True)) + m


def _sims_body(qt, feat_ref, logit_ref, blog_ref, bfeas_ref, sims_ref,
               cmax_ref, rmin_ref, rmax_ref, energy_ref):
    lse = _logsumexp_rows(blog_ref[...])
    bf = bfeas_ref[...]
    bnorm = jnp.sqrt(jnp.sum(bf * bf, axis=1, keepdims=True))
    g = (bf / bnorm + 1e-10) * lse
    f = feat_ref[...]
    norm = jnp.sqrt(jnp.sum(f * f, axis=1, keepdims=True))
    fn = f / norm + 1e-10
    s = lax.dot_general(fn, g, (((1,), (1,)), ((), ())),
                        preferred_element_type=jnp.float32)
    j = pl.program_id(1)

    def emit(s_out, s_for_min):
        s3 = s_out.reshape(qt, CPB, CHUNK)
        sims_ref[...] = s3
        cmax_ref[...] = jnp.max(s3, axis=2).reshape(1, qt, CPB)
        pmin = jnp.min(s_for_min, axis=1, keepdims=True)
        pmax = jnp.max(s_out, axis=1, keepdims=True)
        return pmin, pmax

    @pl.when(j == 0)
    def _():
        pmin, pmax = emit(s, s)
        rmin_ref[...] = pmin
        rmax_ref[...] = pmax
        energy_ref[...] = _logsumexp_rows(logit_ref[...])

    @pl.when(jnp.logical_and(j != 0, j != NBLKJ - 1))
    def _():
        pmin, pmax = emit(s, s)
        rmin_ref[...] = jnp.minimum(rmin_ref[...], pmin)
        rmax_ref[...] = jnp.maximum(rmax_ref[...], pmax)

    @pl.when(j == NBLKJ - 1)
    def _():
        lcol = lax.broadcasted_iota(jnp.int32, (qt, BT), 1)
        pad = lcol >= PAD_LOCAL
        pmin, pmax = emit(jnp.where(pad, NEG, s), jnp.where(pad, 3e38, s))
        rmin_ref[...] = jnp.minimum(rmin_ref[...], pmin)
        rmax_ref[...] = jnp.maximum(rmax_ref[...], pmax)


def _sims_stage(feature, logit, bank_feas, bank_logits):
    nq = feature.shape[0]
    qt = min(QT, nq)
    return pl.pallas_call(
        functools.partial(_sims_body, qt),
        grid=(nq // qt, NBLKJ),
        in_specs=[
            pl.BlockSpec((qt, D), lambda q, j: (q, 0)),
            pl.BlockSpec((qt, NCLS), lambda q, j: (q, 0)),
            pl.BlockSpec((BT, NCLS), lambda q, j: (j, 0)),
            pl.BlockSpec((BT, D), lambda q, j: (j, 0)),
        ],
        out_specs=[
            pl.BlockSpec((qt, CPB, CHUNK), lambda q, j: (q, j, 0)),
            pl.BlockSpec((1, qt, CPB), lambda q, j: (j, q, 0)),
            pl.BlockSpec((qt, 1), lambda q, j: (q, 0)),
            pl.BlockSpec((qt, 1), lambda q, j: (q, 0)),
            pl.BlockSpec((qt, 1), lambda q, j: (q, 0)),
        ],
        out_shape=[
            jax.ShapeDtypeStruct((nq, NCHUNK, CHUNK), jnp.float32),
            jax.ShapeDtypeStruct((NBLKJ, nq, CPB), jnp.float32),
            jax.ShapeDtypeStruct((nq, 1), jnp.float32),
            jax.ShapeDtypeStruct((nq, 1), jnp.float32),
            jax.ShapeDtypeStruct((nq, 1), jnp.float32),
        ],
    )(feature, logit, bank_logits, bank_feas)


def _suffix_select(hcnt, hsum, target):
    lane_f = lax.iota(jnp.int32, LANES).astype(jnp.float32)
    with_sum = hsum is not None

    def cond(carry):
        return jnp.logical_and(jnp.logical_not(carry[3]), carry[0] >= 0)

    def body(carry):
        j, r_c, r_s, done = carry
        c = hcnt[pl.ds(j * LANES, LANES)]
        tot_c = jnp.sum(c)
        cross = r_c + tot_c >= target
        if with_sum:
            tot_s = jnp.sum(hsum[pl.ds(j * LANES, LANES)])
            r_s = jnp.where(cross, r_s, r_s + tot_s)
        return (jnp.where(cross, j, j - 1),
                jnp.where(cross, r_c, r_c + tot_c), r_s, cross)

    init = (jnp.int32(NBINS // LANES - 1), jnp.float32(0.0), jnp.float32(0.0),
            False)
    j, r_c, r_s, _ = lax.while_loop(cond, body, init)

    j = jnp.maximum(j, 0)
    c = hcnt[pl.ds(j * LANES, LANES)]
    rc = lax.rev(jnp.cumsum(lax.rev(c, (0,))), (0,)) + r_c
    m = rc >= target
    lane = jnp.sum(jnp.where(m, 1.0, 0.0)) - 1.0
    sel = lane_f == lane
    c_l = jnp.sum(jnp.where(sel, c, 0.0))
    rc_l = jnp.sum(jnp.where(sel, rc, 0.0))
    b_sel = (j * LANES).astype(jnp.float32) + lane
    cc = rc_l - c_l
    ss = jnp.float32(0.0)
    if with_sum:
        s = hsum[pl.ds(j * LANES, LANES)]
        rs = lax.rev(jnp.cumsum(lax.rev(s, (0,))), (0,)) + r_s
        s_l = jnp.sum(jnp.where(sel, s, 0.0))
        rs_l = jnp.sum(jnp.where(sel, rs, 0.0))
        ss = rs_l - s_l
    return b_sel, cc, ss


def _scalar_at(ref, i, lane_i):
    vbase = (i // LANES) * LANES
    vec = ref[pl.ds(vbase, LANES)]
    sel = lane_i == (i - vbase)
    return jnp.sum(jnp.where(sel, vec, 0.0))


def _sc_topk_body(k_sel, rpt, sims2_hbm, cmax_hbm, lo_hbm, scale_hbm, w1_hbm,
                  esc_hbm, out_hbm,
                  cm_v, idx_v, cand_a, hcnt, hsum, acc_v,
                  lo_v, scale_v, w1_v, esc_v, res_v, sem_a):
    wid = lax.axis_index("s") * NC + lax.axis_index("c")
    base = wid * rpt
    pltpu.sync_copy(lo_hbm.at[pl.ds(base, rpt)], lo_v)
    pltpu.sync_copy(scale_hbm.at[pl.ds(base, rpt)], scale_v)
    pltpu.sync_copy(w1_hbm.at[pl.ds(base, rpt)], w1_v)
    pltpu.sync_copy(esc_hbm.at[pl.ds(base, rpt)], esc_v)
    ones = jnp.full((LANES,), 1.0, jnp.float32)
    zeros = jnp.zeros((LANES,), jnp.float32)
    izeros = jnp.zeros((LANES,), jnp.int32)
    lane_i = lax.iota(jnp.int32, LANES)
    kf = jnp.float32(k_sel)

    @plsc.parallel_loop(0, IDXBUF // LANES, unroll=5)
    def _init_idx(i):
        idx_v[pl.ds(i * LANES, LANES)] = izeros

    def zero_cnt():
        @plsc.parallel_loop(0, NBINS // LANES, unroll=8)
        def _z(i):
            hcnt[pl.ds(i * LANES, LANES)] = zeros

    def row_body(r, carry):
        q = base + r
        pltpu.sync_copy(cmax_hbm.at[:, q], cm_v)
        lo = _scalar_at(lo_v, r, lane_i)
        scale1 = _scalar_at(scale_v, r, lane_i)

        zero_cnt()

        @plsc.parallel_loop(0, NCHUNK // LANES, unroll=7)
        def _pa(i):
            v = cm_v[i, pl.ds(0, LANES)]
            x = jnp.clip((v - lo) * scale1, 0.0, float(NBINS - 1))
            plsc.addupdate_scatter(hcnt, [x.astype(jnp.int32)], ones)

        bA, _, _ = _suffix_select(hcnt, None, kf)
        bAi = bA.astype(jnp.int32)

        def comp(i, off):
            v = cm_v[i, pl.ds(0, LANES)]
            x = jnp.clip((v - lo) * scale1, 0.0, float(NBINS - 1))
            m = x.astype(jnp.int32) >= bAi
            ids = (q * NCHUNK + i * LANES) + lane_i
            plsc.store_compressed(idx_v.at[pl.ds(off, LANES)], ids, mask=m)
            cnt = plsc.all_reduce_population_count(m)
            return off + cnt[0]

        n_surv = lax.fori_loop(0, NCHUNK // LANES, comp, jnp.int32(0))
        nvals = n_surv * CHUNK
        ngr = (n_surv + jnp.int32(GCH - 1)) >> 4

        def wave_fire(wbase, gcount):
            def fire(g, c):
                idxg = idx_v[pl.ds((wbase + g) * LANES, LANES)]
                pltpu.make_async_copy(
                    sims2_hbm.at[idxg],
                    cand_a.at[pl.ds(g * LANES, LANES)], sem_a).start()
                return c

            lax.fori_loop(0, gcount, fire, 0)

            def drain(g, c):
                idxg = idx_v[pl.ds((wbase + g) * LANES, LANES)]
                pltpu.make_async_copy(
                    sims2_hbm.at[idxg],
                    cand_a.at[pl.ds(g * LANES, LANES)], sem_a).wait()
                return c

            lax.fori_loop(0, gcount, drain, 0)

        def proc_wave(wbase, gcount, body):
            wv = wbase * (LANES * CHUNK)

            @plsc.parallel_loop(0, gcount * (GCH * CHUNK // LANES), unroll=8)
            def _p(i):
                row = i >> 3
                col = (i & 7) * LANES
                v = cand_a[row, pl.ds(col, LANES)]
                valm = (wv + i * LANES + lane_i) < nvals
                body(v, valm)

        nwav = (ngr + jnp.int32(WGR - 1)) >> 4

        def waves(body):
            def wv(w, c):
                wbase = w * WGR
                gcount = jnp.minimum(ngr - wbase, WGR)
                wave_fire(wbase, gcount)
                proc_wave(wbase, gcount, body)
                return c

            lax.fori_loop(0, nwav, wv, 0)

        zero_cnt()

        def body_b(v, valm):
            x = jnp.clip((v - lo) * scale1, 0.0, float(NBINS - 1))
            plsc.addupdate_scatter(hcnt, [x.astype(jnp.int32)], ones,
                                   mask=valm)

        waves(body_b)
        b1, cc1, _ = _suffix_select(hcnt, None, kf)
        w1 = _scalar_at(w1_v, r, lane_i)
        blo = lo + b1 * w1
        scale2 = scale1 * jnp.float32(NBINS)
        k1 = kf - cc1
        b1i = b1.astype(jnp.int32)

        zero_cnt()

        @plsc.parallel_loop(0, NBINS // LANES, unroll=8)
        def _zs(i):
            hsum[pl.ds(i * LANES, LANES)] = zeros

        acc_v[pl.ds(0, LANES)] = zeros

        def body_c(v, valm):
            x = jnp.clip((v - lo) * scale1, 0.0, float(NBINS - 1))
            idx1 = x.astype(jnp.int32)
            plsc.addupdate_scatter(
                acc_v, [lane_i],
                jnp.where(jnp.logical_and(valm, idx1 > b1i), v, 0.0))
            msk = jnp.logical_and(valm, idx1 == b1i)
            x2 = jnp.clip((v - blo) * scale2, 0.0, float(NBINS - 1))
            idx2 = x2.astype(jnp.int32)
            plsc.addupdate_scatter(hcnt, [idx2], ones, mask=msk)
            plsc.addupdate_scatter(hsum, [idx2], v, mask=msk)

        @pl.when(nwav == 1)
        def _():
            proc_wave(0, ngr, body_c)

        @pl.when(nwav > 1)
        def _():
            waves(body_c)

        s1 = jnp.sum(acc_v[pl.ds(0, LANES)])
        b2, cc2, s2 = _suffix_select(hcnt, hsum, k1)
        t_hat = blo + b2 * (w1 * jnp.float32(1.0 / NBINS))
        tsum = s1 + s2 + (k1 - cc2) * t_hat
        res = tsum * _scalar_at(esc_v, r, lane_i)
        vbase = (r // LANES) * LANES
        sel = lane_i == (r - vbase)
        plsc.store_scatter(res_v, [jnp.full((LANES,), vbase, jnp.int32) + lane_i],
                           jnp.full((LANES,), 1.0, jnp.float32) * res, mask=sel)
        return carry

    lax.fori_loop(0, rpt, row_body, 0)
    pltpu.sync_copy(res_v, out_hbm.at[pl.ds(base, rpt)])


def _sc_topk(sims2, cmax, lo, scale1, w1, esc, k_sel):
    nq = lo.shape[0]
    rpt = nq // NTEC
    mesh = plsc.VectorSubcoreMesh(core_axis_name="c", subcore_axis_name="s")
    fn = pl.kernel(
        functools.partial(_sc_topk_body, k_sel, rpt),
        mesh=mesh,
        compiler_params=pltpu.CompilerParams(needs_layout_passes=False),
        out_type=jax.ShapeDtypeStruct((nq,), jnp.float32),
        scratch_types=[
            pltpu.VMEM((NBLKJ, CPB), jnp.float32),
            pltpu.VMEM((IDXBUF,), jnp.int32),
            pltpu.VMEM((WGR * GCH, CHUNK), jnp.float32),
            pltpu.VMEM((NBINS,), jnp.float32),
            pltpu.VMEM((NBINS,), jnp.float32),
            pltpu.VMEM((LANES,), jnp.float32),
            pltpu.VMEM((rpt,), jnp.float32),
            pltpu.VMEM((rpt,), jnp.float32),
            pltpu.VMEM((rpt,), jnp.float32),
            pltpu.VMEM((rpt,), jnp.float32),
            pltpu.VMEM((rpt,), jnp.float32),
            pltpu.SemaphoreType.DMA,
        ],
    )
    return fn(sims2, cmax, lo, scale1, w1, esc)


def kernel(feature, logit, bank_feas, bank_logits, k):
    k_sel = logit.shape[-1]
    nsplit = 1
    nqh = NQ // nsplit
    outs = []
    for p in range(nsplit):
        fh = feature[p * nqh:(p + 1) * nqh]
        lh = logit[p * nqh:(p + 1) * nqh]
        sims3, cmax, rmin, rmax, energy = _sims_stage(fh, lh, bank_feas,
                                                      bank_logits)
        sims2 = sims3.reshape(nqh * NCHUNK, CHUNK)
        lo = rmin.reshape(nqh)
        span = jnp.maximum(rmax.reshape(nqh) - lo, 1e-30)
        scale1 = jnp.float32(NBINS) / span
        w1 = span * jnp.float32(1.0 / NBINS)
        esc = -energy.reshape(nqh) / k
        outs.append(_sc_topk(sims2, cmax, lo, scale1, w1, esc, k_sel))
    return jnp.concatenate(outs)
